# baseline (speedup 1.0000x reference)
import dataclasses
import functools

import jax
import jax.numpy as jnp
from jax import lax
from jax.experimental import pallas as pl
from jax.experimental.pallas import tpu as pltpu
from jax.experimental.pallas import tpu_sc as plsc

N_NODES = 10000
D = 128
NP = 10240
NC, NS = 2, 16
NW = NC * NS

B = 128
E_BATCHES = 80
E_PAD = NW * E_BATCHES * B
NID_BATCHES = 2
NID_PAD = NW * NID_BATCHES * B

CH = 8
EB_CORE = (112, 48)

ROWS_PER_TILE = NP // NS

_mesh = plsc.VectorSubcoreMesh(core_axis_name="c", subcore_axis_name="s")
_f32 = jnp.float32

_sc_params = pltpu.CompilerParams()
if "needs_layout_passes" in pltpu.CompilerParams.__dataclass_fields__:
    _sc_params = dataclasses.replace(_sc_params, needs_layout_passes=False)


@functools.partial(
    pl.kernel,
    mesh=_mesh,
    out_type=(
        jax.ShapeDtypeStruct((NW, NP), _f32),
        jax.ShapeDtypeStruct((NW, NP), _f32),
    ),
    scratch_types=[
        pltpu.VMEM((E_BATCHES, B), jnp.int32),
        pltpu.VMEM((NID_BATCHES, B), jnp.int32),
        pltpu.VMEM((NP,), _f32),
        pltpu.VMEM((NP,), _f32),
    ],
    compiler_params=_sc_params,
)
def _hist_kernel(row_hbm, nid_hbm, deg_out, cnt_out,
                 idx_v, nidx_v, hist_deg, hist_cnt):
    cid = lax.axis_index("c")
    sid = lax.axis_index("s")
    wid = sid * NC + cid

    zeros16 = jnp.zeros((16,), _f32)

    @pl.loop(0, NP, step=16)
    def _(i):
        hist_deg[pl.ds(i, 16)] = zeros16
        hist_cnt[pl.ds(i, 16)] = zeros16

    pltpu.sync_copy(row_hbm.at[pl.ds(wid * E_BATCHES, E_BATCHES)], idx_v)
    pltpu.sync_copy(nid_hbm.at[pl.ds(wid * NID_BATCHES, NID_BATCHES)], nidx_v)

    @pl.loop(0, E_BATCHES)
    def _(j):
        for k in range(B // 16):
            v = idx_v[j, pl.ds(k * 16, 16)]
            run, last = plsc.scan_count(v)
            plsc.addupdate_scatter(
                hist_deg, [v], run.astype(_f32), mask=last)

    for j in range(NID_BATCHES):
        for k in range(B // 16):
            v = nidx_v[j, pl.ds(k * 16, 16)]
            run, last = plsc.scan_count(v)
            plsc.addupdate_scatter(
                hist_cnt, [v], run.astype(_f32), mask=last)

    pltpu.sync_copy(hist_deg, deg_out.at[wid])
    pltpu.sync_copy(hist_cnt, cnt_out.at[wid])


@functools.partial(
    pl.kernel,
    mesh=_mesh,
    out_type=jax.ShapeDtypeStruct((NC, NP, D), _f32),
    scratch_types=[
        pltpu.VMEM((CH, B), jnp.int32),
        pltpu.VMEM((CH, B), jnp.int32),
        pltpu.VMEM((CH, B), jnp.int32),
        pltpu.VMEM((CH, B), jnp.int32),
        pltpu.VMEM((B, D), _f32),
        pltpu.VMEM((B, D), _f32),
        pltpu.VMEM_SHARED((NP, D), _f32),
        pltpu.SemaphoreType.DMA,
        pltpu.SemaphoreType.DMA,
        pltpu.SemaphoreType.DMA,
    ],
)
def _scatter_kernel(y_hbm, y2_hbm, row_hbm, col_hbm, acc_out,
                    idxr0, idxr1, idxc0, idxc1, rows0, rows1, acc,
                    sem0, sem1, semi):
    cid = lax.axis_index("c")
    sid = lax.axis_index("s")

    idxr = (idxr0, idxr1)
    idxc = (idxc0, idxc1)

    @pl.loop(0, B)
    def _(i):
        for k in range(D // 16):
            rows0[i, pl.ds(k * 16, 16)] = jnp.zeros((16,), _f32)

    base = sid * ROWS_PER_TILE
    for m in range(ROWS_PER_TILE // B):
        pltpu.sync_copy(rows0, acc.at[pl.ds(base + m * B, B)])
    plsc.subcore_barrier()

    def pipeline(eb, ebase, y_hbm):
        nch = eb // CH

        def load_chunk_async(c):
            pltpu.async_copy(
                row_hbm.at[pl.ds(ebase + c * CH, CH)], idxr[c % 2], semi)
            pltpu.async_copy(
                col_hbm.at[pl.ds(ebase + c * CH, CH)], idxc[c % 2], semi)

        def wait_chunk(c):
            pltpu.make_async_copy(
                row_hbm.at[pl.ds(ebase + c * CH, CH)], idxr[c % 2], semi).wait()
            pltpu.make_async_copy(
                col_hbm.at[pl.ds(ebase + c * CH, CH)], idxc[c % 2], semi).wait()

        load_chunk_async(0)
        wait_chunk(0)
        if nch > 1:
            load_chunk_async(1)
        pltpu.async_copy(y_hbm.at[idxr0.at[0]], rows0, sem0)
        pltpu.async_copy(y_hbm.at[idxr0.at[1]], rows1, sem1)

        for c in range(nch):
            cur_r, cur_c = idxr[c % 2], idxc[c % 2]
            if 2 <= c + 1 < nch:
                load_chunk_async(c + 1)

            @pl.loop(0, CH - 2, step=2)
            def _(j, cur_r=cur_r, cur_c=cur_c):
                pltpu.make_async_copy(
                    y_hbm.at[cur_r.at[j]], rows0, sem0).wait()
                pltpu.sync_copy(rows0, acc.at[cur_c.at[j]], add=True)
                pltpu.async_copy(y_hbm.at[cur_r.at[j + 2]], rows0, sem0)

                pltpu.make_async_copy(
                    y_hbm.at[cur_r.at[j + 1]], rows1, sem1).wait()
                pltpu.sync_copy(rows1, acc.at[cur_c.at[j + 1]], add=True)
                pltpu.async_copy(y_hbm.at[cur_r.at[j + 3]], rows1, sem1)

            if c + 1 < nch:
                wait_chunk(c + 1)

            pltpu.make_async_copy(y_hbm.at[cur_r.at[CH - 2]], rows0, sem0).wait()
            pltpu.sync_copy(rows0, acc.at[cur_c.at[CH - 2]], add=True)
            if c + 1 < nch:
                pltpu.async_copy(y_hbm.at[idxr[(c + 1) % 2].at[0]], rows0, sem0)

            pltpu.make_async_copy(y_hbm.at[cur_r.at[CH - 1]], rows1, sem1).wait()
            pltpu.sync_copy(rows1, acc.at[cur_c.at[CH - 1]], add=True)
            if c + 1 < nch:
                pltpu.async_copy(y_hbm.at[idxr[(c + 1) % 2].at[1]], rows1, sem1)

    eb0, eb1 = EB_CORE

    if eb0 > 0:
        @pl.when(cid == 0)
        def _():
            pipeline(eb0, sid * (eb0 + eb1), y_hbm)

    if eb1 > 0:
        @pl.when(cid == 1)
        def _():
            pipeline(eb1, sid * (eb0 + eb1) + eb0, y2_hbm)

    plsc.subcore_barrier()
    pltpu.sync_copy(acc.at[pl.ds(base, ROWS_PER_TILE)],
                    acc_out.at[cid, pl.ds(base, ROWS_PER_TILE)])


def _transform_body(x_ref, w_ref, wid_ref, degp_ref, cntp_ref, y_ref, y2_ref):
    deg = jnp.sum(degp_ref[...], axis=0) + 1.0
    cnt = jnp.sum(cntp_ref[...], axis=0)
    dis = lax.rsqrt(deg)
    dn = (((1,), (0,)), ((), ()))
    xw = lax.dot_general(x_ref[...], w_ref[...], dn,
                         precision=lax.Precision.HIGHEST,
                         preferred_element_type=_f32)
    xid = lax.dot_general(x_ref[...], wid_ref[...], dn,
                          precision=lax.Precision.HIGHEST,
                          preferred_element_type=_f32)
    y = dis[:, None] * (xw + cnt[:, None] * xid)
    y_ref[...] = y
    y2_ref[...] = y


_RB = 1024


def _transform(x_pad, w, w_id, deg_p, cnt_p):
    return pl.pallas_call(
        _transform_body,
        grid=(NP // _RB,),
        in_specs=[
            pl.BlockSpec((_RB, D), lambda i: (i, 0)),
            pl.BlockSpec((D, D), lambda i: (0, 0)),
            pl.BlockSpec((D, D), lambda i: (0, 0)),
            pl.BlockSpec((NW, _RB), lambda i: (0, i)),
            pl.BlockSpec((NW, _RB), lambda i: (0, i)),
        ],
        out_specs=[pl.BlockSpec((_RB, D), lambda i: (i, 0)),
                   pl.BlockSpec((_RB, D), lambda i: (i, 0))],
        out_shape=[jax.ShapeDtypeStruct((NP, D), _f32),
                   jax.ShapeDtypeStruct((NP, D), _f32)],
    )(x_pad, w, w_id, deg_p, cnt_p)


def _final_body(acc_ref, y_ref, degp_ref, o_ref):
    deg = jnp.sum(degp_ref[...], axis=0) + 1.0
    dis = lax.rsqrt(deg)
    o_ref[...] = dis[:, None] * (acc_ref[0] + acc_ref[1] + y_ref[...])


def _final(acc, y, deg_p):
    return pl.pallas_call(
        _final_body,
        grid=(NP // _RB,),
        in_specs=[
            pl.BlockSpec((NC, _RB, D), lambda i: (0, i, 0)),
            pl.BlockSpec((_RB, D), lambda i: (i, 0)),
            pl.BlockSpec((NW, _RB), lambda i: (0, i)),
        ],
        out_specs=pl.BlockSpec((_RB, D), lambda i: (i, 0)),
        out_shape=jax.ShapeDtypeStruct((NP, D), _f32),
    )(acc, y, deg_p)


def kernel(x, edge_index, node_id, weight, weight_id):
    ei = edge_index.astype(jnp.int32)
    nid = node_id.astype(jnp.int32)

    row_pad = jnp.full((E_PAD,), N_NODES, jnp.int32).at[: ei.shape[1]].set(ei[0])
    col_pad = jnp.zeros((E_PAD,), jnp.int32).at[: ei.shape[1]].set(ei[1])
    row2 = row_pad.reshape(NW * E_BATCHES, B)
    col2 = col_pad.reshape(NW * E_BATCHES, B)
    nid2 = (jnp.full((NID_PAD,), N_NODES, jnp.int32)
            .at[: nid.shape[0]].set(nid).reshape(NW * NID_BATCHES, B))
    x_pad = jnp.zeros((NP, D), _f32).at[:N_NODES].set(x)

    deg_p, cnt_p = _hist_kernel(row2, nid2)
    y, y2 = _transform(x_pad, weight, weight_id, deg_p, cnt_p)
    acc = _scatter_kernel(y, y2, row2, col2)
    out = _final(acc, y, deg_p)
    return out[:N_NODES]


# speedup vs baseline: 1.0290x; 1.0290x over previous
import dataclasses
import functools

import jax
import jax.numpy as jnp
from jax import lax
from jax.experimental import pallas as pl
from jax.experimental.pallas import tpu as pltpu
from jax.experimental.pallas import tpu_sc as plsc

N_NODES = 10000
D = 128
NP = 10240
NC, NS = 2, 16
NW = NC * NS

B = 128
E_BATCHES = 80
E_PAD = NW * E_BATCHES * B
NID_BATCHES = 2
NID_PAD = NW * NID_BATCHES * B

CH = 8
EB_CORE = (120, 40)

ROWS_PER_TILE = NP // NS

_mesh = plsc.VectorSubcoreMesh(core_axis_name="c", subcore_axis_name="s")
_f32 = jnp.float32

_sc_params = pltpu.CompilerParams()
if "needs_layout_passes" in pltpu.CompilerParams.__dataclass_fields__:
    _sc_params = dataclasses.replace(_sc_params, needs_layout_passes=False)


@functools.partial(
    pl.kernel,
    mesh=_mesh,
    out_type=(
        jax.ShapeDtypeStruct((NW, NP), _f32),
        jax.ShapeDtypeStruct((NW, NP), _f32),
    ),
    scratch_types=[
        pltpu.VMEM((E_BATCHES, B), jnp.int32),
        pltpu.VMEM((NID_BATCHES, B), jnp.int32),
        pltpu.VMEM((NP,), _f32),
        pltpu.VMEM((NP,), _f32),
    ],
    compiler_params=_sc_params,
)
def _hist_kernel(row_hbm, nid_hbm, deg_out, cnt_out,
                 idx_v, nidx_v, hist_deg, hist_cnt):
    cid = lax.axis_index("c")
    sid = lax.axis_index("s")
    wid = sid * NC + cid

    zeros16 = jnp.zeros((16,), _f32)

    @pl.loop(0, NP, step=16)
    def _(i):
        hist_deg[pl.ds(i, 16)] = zeros16
        hist_cnt[pl.ds(i, 16)] = zeros16

    pltpu.sync_copy(row_hbm.at[pl.ds(wid * E_BATCHES, E_BATCHES)], idx_v)
    pltpu.sync_copy(nid_hbm.at[pl.ds(wid * NID_BATCHES, NID_BATCHES)], nidx_v)

    @pl.loop(0, E_BATCHES)
    def _(j):
        for k in range(B // 16):
            v = idx_v[j, pl.ds(k * 16, 16)]
            run, last = plsc.scan_count(v)
            plsc.addupdate_scatter(
                hist_deg, [v], run.astype(_f32), mask=last)

    for j in range(NID_BATCHES):
        for k in range(B // 16):
            v = nidx_v[j, pl.ds(k * 16, 16)]
            run, last = plsc.scan_count(v)
            plsc.addupdate_scatter(
                hist_cnt, [v], run.astype(_f32), mask=last)

    pltpu.sync_copy(hist_deg, deg_out.at[wid])
    pltpu.sync_copy(hist_cnt, cnt_out.at[wid])


@functools.partial(
    pl.kernel,
    mesh=_mesh,
    out_type=jax.ShapeDtypeStruct((NC, NP, D), _f32),
    scratch_types=[
        pltpu.VMEM((CH, B), jnp.int32),
        pltpu.VMEM((CH, B), jnp.int32),
        pltpu.VMEM((CH, B), jnp.int32),
        pltpu.VMEM((CH, B), jnp.int32),
        pltpu.VMEM((B, D), _f32),
        pltpu.VMEM((B, D), _f32),
        pltpu.VMEM_SHARED((NP, D), _f32),
        pltpu.SemaphoreType.DMA,
        pltpu.SemaphoreType.DMA,
        pltpu.SemaphoreType.DMA,
    ],
)
def _scatter_kernel(y_hbm, y2_hbm, row_hbm, col_hbm, acc_out,
                    idxr0, idxr1, idxc0, idxc1, rows0, rows1, acc,
                    sem0, sem1, semi):
    cid = lax.axis_index("c")
    sid = lax.axis_index("s")

    idxr = (idxr0, idxr1)
    idxc = (idxc0, idxc1)

    @pl.loop(0, B)
    def _(i):
        for k in range(D // 16):
            rows0[i, pl.ds(k * 16, 16)] = jnp.zeros((16,), _f32)

    base = sid * ROWS_PER_TILE
    for m in range(ROWS_PER_TILE // B):
        pltpu.sync_copy(rows0, acc.at[pl.ds(base + m * B, B)])
    plsc.subcore_barrier()

    def pipeline(eb, ebase, y_hbm):
        nch = eb // CH

        def load_chunk_async(c):
            pltpu.async_copy(
                row_hbm.at[pl.ds(ebase + c * CH, CH)], idxr[c % 2], semi)
            pltpu.async_copy(
                col_hbm.at[pl.ds(ebase + c * CH, CH)], idxc[c % 2], semi)

        def wait_chunk(c):
            pltpu.make_async_copy(
                row_hbm.at[pl.ds(ebase + c * CH, CH)], idxr[c % 2], semi).wait()
            pltpu.make_async_copy(
                col_hbm.at[pl.ds(ebase + c * CH, CH)], idxc[c % 2], semi).wait()

        load_chunk_async(0)
        wait_chunk(0)
        if nch > 1:
            load_chunk_async(1)
        pltpu.async_copy(y_hbm.at[idxr0.at[0]], rows0, sem0)
        pltpu.async_copy(y_hbm.at[idxr0.at[1]], rows1, sem1)

        for c in range(nch):
            cur_r, cur_c = idxr[c % 2], idxc[c % 2]
            if 2 <= c + 1 < nch:
                load_chunk_async(c + 1)

            @pl.loop(0, CH - 2, step=2)
            def _(j, cur_r=cur_r, cur_c=cur_c):
                pltpu.make_async_copy(
                    y_hbm.at[cur_r.at[j]], rows0, sem0).wait()
                pltpu.sync_copy(rows0, acc.at[cur_c.at[j]], add=True)
                pltpu.async_copy(y_hbm.at[cur_r.at[j + 2]], rows0, sem0)

                pltpu.make_async_copy(
                    y_hbm.at[cur_r.at[j + 1]], rows1, sem1).wait()
                pltpu.sync_copy(rows1, acc.at[cur_c.at[j + 1]], add=True)
                pltpu.async_copy(y_hbm.at[cur_r.at[j + 3]], rows1, sem1)

            if c + 1 < nch:
                wait_chunk(c + 1)

            pltpu.make_async_copy(y_hbm.at[cur_r.at[CH - 2]], rows0, sem0).wait()
            pltpu.sync_copy(rows0, acc.at[cur_c.at[CH - 2]], add=True)
            if c + 1 < nch:
                pltpu.async_copy(y_hbm.at[idxr[(c + 1) % 2].at[0]], rows0, sem0)

            pltpu.make_async_copy(y_hbm.at[cur_r.at[CH - 1]], rows1, sem1).wait()
            pltpu.sync_copy(rows1, acc.at[cur_c.at[CH - 1]], add=True)
            if c + 1 < nch:
                pltpu.async_copy(y_hbm.at[idxr[(c + 1) % 2].at[1]], rows1, sem1)

    eb0, eb1 = EB_CORE

    if eb0 > 0:
        @pl.when(cid == 0)
        def _():
            pipeline(eb0, sid * (eb0 + eb1), y_hbm)

    if eb1 > 0:
        @pl.when(cid == 1)
        def _():
            pipeline(eb1, sid * (eb0 + eb1) + eb0, y2_hbm)

    plsc.subcore_barrier()
    pltpu.sync_copy(acc.at[pl.ds(base, ROWS_PER_TILE)],
                    acc_out.at[cid, pl.ds(base, ROWS_PER_TILE)])


def _transform_body(x_ref, w_ref, wid_ref, degp_ref, cntp_ref, y_ref, y2_ref):
    deg = jnp.sum(degp_ref[...], axis=0) + 1.0
    cnt = jnp.sum(cntp_ref[...], axis=0)
    dis = lax.rsqrt(deg)
    dn = (((1,), (0,)), ((), ()))
    xw = lax.dot_general(x_ref[...], w_ref[...], dn,
                         precision=lax.Precision.HIGHEST,
                         preferred_element_type=_f32)
    xid = lax.dot_general(x_ref[...], wid_ref[...], dn,
                          precision=lax.Precision.HIGHEST,
                          preferred_element_type=_f32)
    y = dis[:, None] * (xw + cnt[:, None] * xid)
    y_ref[...] = y
    y2_ref[...] = y


_RB = 1024


def _transform(x_pad, w, w_id, deg_p, cnt_p):
    return pl.pallas_call(
        _transform_body,
        grid=(NP // _RB,),
        in_specs=[
            pl.BlockSpec((_RB, D), lambda i: (i, 0)),
            pl.BlockSpec((D, D), lambda i: (0, 0)),
            pl.BlockSpec((D, D), lambda i: (0, 0)),
            pl.BlockSpec((NW, _RB), lambda i: (0, i)),
            pl.BlockSpec((NW, _RB), lambda i: (0, i)),
        ],
        out_specs=[pl.BlockSpec((_RB, D), lambda i: (i, 0)),
                   pl.BlockSpec((_RB, D), lambda i: (i, 0))],
        out_shape=[jax.ShapeDtypeStruct((NP, D), _f32),
                   jax.ShapeDtypeStruct((NP, D), _f32)],
    )(x_pad, w, w_id, deg_p, cnt_p)


def _final_body(acc_ref, y_ref, degp_ref, o_ref):
    deg = jnp.sum(degp_ref[...], axis=0) + 1.0
    dis = lax.rsqrt(deg)
    o_ref[...] = dis[:, None] * (acc_ref[0] + acc_ref[1] + y_ref[...])


def _final(acc, y, deg_p):
    return pl.pallas_call(
        _final_body,
        grid=(NP // _RB,),
        in_specs=[
            pl.BlockSpec((NC, _RB, D), lambda i: (0, i, 0)),
            pl.BlockSpec((_RB, D), lambda i: (i, 0)),
            pl.BlockSpec((NW, _RB), lambda i: (0, i)),
        ],
        out_specs=pl.BlockSpec((_RB, D), lambda i: (i, 0)),
        out_shape=jax.ShapeDtypeStruct((NP, D), _f32),
    )(acc, y, deg_p)


def kernel(x, edge_index, node_id, weight, weight_id):
    ei = edge_index.astype(jnp.int32)
    nid = node_id.astype(jnp.int32)

    row_pad = jnp.full((E_PAD,), N_NODES, jnp.int32).at[: ei.shape[1]].set(ei[0])
    col_pad = jnp.zeros((E_PAD,), jnp.int32).at[: ei.shape[1]].set(ei[1])
    row2 = row_pad.reshape(NW * E_BATCHES, B)
    col2 = col_pad.reshape(NW * E_BATCHES, B)
    nid2 = (jnp.full((NID_PAD,), N_NODES, jnp.int32)
            .at[: nid.shape[0]].set(nid).reshape(NW * NID_BATCHES, B))
    x_pad = jnp.zeros((NP, D), _f32).at[:N_NODES].set(x)

    deg_p, cnt_p = _hist_kernel(row2, nid2)
    y, y2 = _transform(x_pad, weight, weight_id, deg_p, cnt_p)
    acc = _scatter_kernel(y, y2, row2, col2)
    out = _final(acc, y, deg_p)
    return out[:N_NODES]
